# baseline (speedup 1.0000x reference)
import functools

import jax
import jax.numpy as jnp
from jax import lax
from jax.experimental import pallas as pl
from jax.experimental.pallas import tpu as pltpu
from jax.experimental.pallas import tpu_sc as plsc

B = 16384
D = 64
OUT_D = 4 * D
N_NODES = 1000000
NUM_CORES = 2
NUM_SUBCORES = 16
NW = NUM_CORES * NUM_SUBCORES

WCOLS = 31232
WIN = 512
NWIN = WCOLS // WIN
SUB = 4096
NSUB = 8
CAP1 = 704
CAP2 = 160
NROW = 192
IDXC = 4096
DUMP = B
TAILA = NW * WCOLS
TAILB = TAILA + WIN


def _iota16():
    return lax.broadcasted_iota(jnp.int32, (16,), 0)


@functools.partial(
    pl.kernel,
    mesh=plsc.VectorSubcoreMesh(core_axis_name="c", subcore_axis_name="s"),
    out_type=jax.ShapeDtypeStruct((B + 16, 128), jnp.float32),
    scratch_types=[
        pltpu.VMEM((IDXC,), jnp.int32),
        pltpu.VMEM((CAP1 + 16,), jnp.int32),
        pltpu.VMEM((CAP1 + 16,), jnp.int32),
        pltpu.VMEM((CAP2 + 16,), jnp.int32),
        pltpu.VMEM((CAP2 + 16,), jnp.int32),
        pltpu.VMEM((CAP2 + 16,), jnp.int32),
        pltpu.VMEM((CAP2 + 16,), jnp.int32),
        pltpu.VMEM((D, WIN), jnp.float32),
        pltpu.VMEM((D, WIN), jnp.float32),
        pltpu.VMEM((D, D), jnp.float32),
        pltpu.VMEM((NROW, 128), jnp.float32),
        pltpu.VMEM((NROW, 128), jnp.float32),
        pltpu.VMEM((128,), jnp.int32),
        pltpu.VMEM((64,), jnp.int32),
        pltpu.VMEM((128,), jnp.int32),
        pltpu.VMEM((64,), jnp.int32),
        pltpu.SemaphoreType.DMA,
        pltpu.SemaphoreType.DMA,
        pltpu.SemaphoreType.DMA,
        pltpu.SemaphoreType.DMA,
    ],
    compiler_params=pltpu.CompilerParams(needs_layout_passes=False),
)
def _sc_stream_gather(idx_hbm, tableT, tailT, pos, idx_v, h1i_v, h1j_v,
                      h2i_v, h2j_v, h3i_v, h3j_v, slabA, slabB, tail_v,
                      rb0, rb1, jlP0, jlQ0, jlP1, jlQ1,
                      semA, semB, semS0, semS1):
    wid = lax.axis_index("s") * NUM_CORES + lax.axis_index("c")
    wbase = wid * WCOLS
    lo = wbase
    hi = jnp.where(wid == NW - 1, N_NODES, wbase + WCOLS)

    for k, (slab, sem) in enumerate([(slabA, semA), (slabB, semB)]):
        col = pl.multiple_of(wbase + k * WIN, WIN)
        pltpu.async_copy(tableT.at[:, pl.ds(col, WIN)], slab, sem)

    cnt1 = 0
    for ch in range(B // IDXC):
        pltpu.sync_copy(idx_hbm.at[pl.ds(ch * IDXC, IDXC)], idx_v)

        def l1_body(i, cnt, ch=ch):
            v = idx_v[pl.ds(i * 16, 16)]
            jv = _iota16() + (i * 16 + ch * IDXC)
            m = (v >= lo) & (v < hi)
            npop = plsc.all_reduce_population_count(m)[0]

            @pl.when(npop > 0)
            def _():
                plsc.store_compressed(h1i_v.at[pl.ds(cnt, 16)], v, mask=m)
                plsc.store_compressed(h1j_v.at[pl.ds(cnt, 16)], jv, mask=m)

            return cnt + npop

        cnt1 = lax.fori_loop(0, IDXC // 16, l1_body, cnt1)

    def sub_count(s):
        slo = wbase + s * SUB
        shi = jnp.minimum(slo + SUB, hi)

        def l2_body(i, cnt):
            v = h1i_v[pl.ds(i * 16, 16)]
            jv = h1j_v[pl.ds(i * 16, 16)]
            m = (((_iota16() + i * 16) < cnt1) & (v >= slo) & (v < shi))
            npop = plsc.all_reduce_population_count(m)[0]

            @pl.when(npop > 0)
            def _():
                plsc.store_compressed(h2i_v.at[pl.ds(cnt, 16)], v, mask=m)
                plsc.store_compressed(h2j_v.at[pl.ds(cnt, 16)], jv, mask=m)

            return cnt + npop

        return lax.fori_loop(0, (cnt1 + 15) >> 4, l2_body, 0)

    def compress_window(gcol_lo, cnt2):
        def l3_body(i, cnt):
            v = h2i_v[pl.ds(i * 16, 16)]
            jv = h2j_v[pl.ds(i * 16, 16)]
            m = (((_iota16() + i * 16) < cnt2)
                 & (v >= gcol_lo) & (v < gcol_lo + WIN))
            npop = plsc.all_reduce_population_count(m)[0]

            @pl.when(npop > 0)
            def _():
                plsc.store_compressed(h3i_v.at[pl.ds(cnt, 16)], v, mask=m)
                plsc.store_compressed(h3j_v.at[pl.ds(cnt, 16)], jv, mask=m)

            return cnt + npop

        return lax.fori_loop(0, (cnt2 + 15) >> 4, l3_body, 0)

    def append_group(slab, fetch_lo, off, cnt3, c, rb, jlP, jlQ):
        hv = h3i_v[pl.ds(off, 16)]
        jv = h3j_v[pl.ds(off, 16)]
        validm = (_iota16() + off) < cnt3
        lvec = jnp.where(validm, hv - fetch_lo, 0)

        def d_body(d, _):
            dv = jnp.full((16,), d, jnp.int32)
            vals = plsc.load_gather(slab, [dv, lvec], mask=validm)
            plsc.store_scatter(rb, [_iota16() + c, dv], vals)
            return 0

        lax.fori_loop(0, D, d_body, 0)
        jdst = jnp.where(validm, jv, DUMP)

        @pl.when(c < 128)
        def _():
            jlP[pl.ds(c, 16)] = jdst

        @pl.when(c >= 128)
        def _():
            jlQ[pl.ds(c - 128, 16)] = jdst

        return c + 16

    def process_window(slab, gcol, fetch_lo, cnt2, c, rb, jlP, jlQ):
        cnt3 = compress_window(gcol, cnt2)
        c = append_group(slab, fetch_lo, 0, cnt3, c, rb, jlP, jlQ)

        def rare(i, cc):
            return append_group(slab, fetch_lo, i * 16, cnt3, cc, rb, jlP,
                                jlQ)

        return lax.fori_loop(1, (cnt3 + 15) >> 4, rare, c)

    rbs = [rb0, rb1]
    jlPs = [jlP0, jlP1]
    jlQs = [jlQ0, jlQ1]
    ssems = [semS0, semS1]

    def fire(par):
        pltpu.async_copy(rbs[par].at[pl.ds(0, 128)],
                         pos.at[jlPs[par]], ssems[par])
        pltpu.async_copy(rbs[par].at[pl.ds(128, 64)],
                         pos.at[jlQs[par]], ssems[par])

    def drain(par):
        pltpu.make_async_copy(rbs[par].at[pl.ds(0, 128)],
                              pos.at[jlPs[par]], ssems[par]).wait()
        pltpu.make_async_copy(rbs[par].at[pl.ds(128, 64)],
                              pos.at[jlQs[par]], ssems[par]).wait()

    def reinit_jl(par):
        dfull = jnp.full((16,), DUMP, jnp.int32)
        for q in range(8):
            jlPs[par][pl.ds(q * 16, 16)] = dfull
        for q in range(4):
            jlQs[par][pl.ds(q * 16, 16)] = dfull

    for par in range(2):
        reinit_jl(par)
        fire(par)

    for s in range(NSUB):
        par = s % 2
        cnt2 = sub_count(s)
        drain(par)
        reinit_jl(par)
        npairs = 4 if s < NSUB - 1 else 2

        def pair_body(p, c, s=s, par=par, cnt2=cnt2):
            g0 = s * 8 + 2 * p
            for (slab, sem, goff) in ((slabA, semA, 0), (slabB, semB, 1)):
                g = g0 + goff
                col = pl.multiple_of(wbase + g * WIN, WIN)
                pltpu.make_async_copy(
                    tableT.at[:, pl.ds(col, WIN)], slab, sem).wait()
                c = process_window(slab, col, col, cnt2, c,
                                   rbs[par], jlPs[par], jlQs[par])

                @pl.when(g + 2 < NWIN)
                def _(slab=slab, sem=sem, g=g):
                    coln = pl.multiple_of(wbase + (g + 2) * WIN, WIN)
                    pltpu.async_copy(
                        tableT.at[:, pl.ds(coln, WIN)], slab, sem)

            return c

        c = lax.fori_loop(0, npairs, pair_body, jnp.int32(0))

        if s == NSUB - 1:
            col = pl.multiple_of(wbase + (NWIN - 1) * WIN, WIN)
            pltpu.make_async_copy(
                tableT.at[:, pl.ds(col, WIN)], slabA, semA).wait()
            c = process_window(slabA, col, col, cnt2, c,
                               rbs[par], jlPs[par], jlQs[par])

            @pl.when(wid == NW - 1)
            def _(c=c, cnt2=cnt2, par=par):
                pltpu.async_copy(
                    tableT.at[:, pl.ds(TAILA, WIN)], slabA, semA).wait()
                ct = process_window(slabA, TAILA, TAILA, cnt2, c,
                                    rbs[par], jlPs[par], jlQs[par])
                pltpu.async_copy(tailT, tail_v, semB).wait()
                cnt3 = compress_window(TAILB, cnt2)

                def tail_groups(i, cc):
                    return append_group(tail_v, TAILB, i * 16, cnt3, cc,
                                        rbs[par], jlPs[par], jlQs[par])

                lax.fori_loop(0, (cnt3 + 15) >> 4, tail_groups, ct)

        fire(par)

    drain(0)
    drain(1)


def _concat_body(z_src_ref, z_dst_ref, pos_ref, t_ref, out_ref):
    out_ref[...] = jnp.concatenate(
        [z_src_ref[...], z_dst_ref[...], pos_ref[...][:, :D], t_ref[...]],
        axis=-1)


_R = 2048
_concat = pl.pallas_call(
    _concat_body,
    grid=(B // _R,),
    in_specs=[pl.BlockSpec((_R, D), lambda i: (i, 0))] * 2
    + [pl.BlockSpec((_R, 128), lambda i: (i, 0))]
    + [pl.BlockSpec((_R, D), lambda i: (i, 0))],
    out_specs=pl.BlockSpec((_R, OUT_D), lambda i: (i, 0)),
    out_shape=jax.ShapeDtypeStruct((B, OUT_D), jnp.float32),
)


def kernel(z_src, z_dst, raw_msg, t_enc, embedding_weight):
    idx = raw_msg.astype(jnp.int32)
    tableT = embedding_weight.T
    tailT = lax.slice(tableT, (0, TAILB), (D, N_NODES))
    pos128 = _sc_stream_gather(idx, tableT, tailT)
    return _concat(z_src, z_dst, pos128, t_enc)


# speedup vs baseline: 6.7159x; 6.7159x over previous
import functools

import jax
import jax.numpy as jnp
from jax import lax
from jax.experimental import pallas as pl
from jax.experimental.pallas import tpu as pltpu
from jax.experimental.pallas import tpu_sc as plsc

B = 16384
D = 64
OUT_D = 4 * D
NUM_CORES = 2
NUM_SUBCORES = 16
NW = NUM_CORES * NUM_SUBCORES
BPW = B // NW
G = 16
NGROUP = BPW // G


@functools.partial(
    pl.kernel,
    mesh=plsc.VectorSubcoreMesh(core_axis_name="c", subcore_axis_name="s"),
    out_type=jax.ShapeDtypeStruct((B, D), jnp.float32),
    scratch_types=[
        pltpu.VMEM((BPW,), jnp.int32),
        pltpu.VMEM((G, 8, D), jnp.float32),
        pltpu.VMEM((BPW, D), jnp.float32),
        pltpu.SemaphoreType.DMA,
    ],
)
def _sc_gather(idx_hbm, table3, out, idx_v, tiles_v, rows_v, sem):
    wid = lax.axis_index("s") * NUM_CORES + lax.axis_index("c")
    base = wid * BPW
    pltpu.sync_copy(idx_hbm.at[pl.ds(base, BPW)], idx_v)

    def group_body(g, _):
        gbase = g * G
        vec = idx_v[pl.ds(gbase, G)]
        tvec = lax.shift_right_logical(vec, 3)
        rvec = vec & 7
        handles = []
        for j in range(G):
            handles.append(pltpu.async_copy(
                table3.at[pl.ds(tvec[j], 1)], tiles_v.at[pl.ds(j, 1)], sem))
        for h in handles:
            h.wait()
        for j in range(G):
            for k in range(D // 16):
                rows_v[gbase + j, pl.ds(k * 16, 16)] = (
                    tiles_v[j, rvec[j], pl.ds(k * 16, 16)])
        return 0

    lax.fori_loop(0, NGROUP, group_body, 0)
    pltpu.sync_copy(rows_v, out.at[pl.ds(base, BPW)])


def _concat_body(z_src_ref, z_dst_ref, pos_ref, t_ref, out_ref):
    out_ref[...] = jnp.concatenate(
        [z_src_ref[...], z_dst_ref[...], pos_ref[...], t_ref[...]], axis=-1)


_R = 2048
_concat = pl.pallas_call(
    _concat_body,
    grid=(B // _R,),
    in_specs=[pl.BlockSpec((_R, D), lambda i: (i, 0))] * 4,
    out_specs=pl.BlockSpec((_R, OUT_D), lambda i: (i, 0)),
    out_shape=jax.ShapeDtypeStruct((B, OUT_D), jnp.float32),
)


def kernel(z_src, z_dst, raw_msg, t_enc, embedding_weight):
    idx = raw_msg.astype(jnp.int32)
    table3 = embedding_weight.reshape(125000, 8, D)
    pos_msg = _sc_gather(idx, table3)
    return _concat(z_src, z_dst, pos_msg, t_enc)


# speedup vs baseline: 6.8357x; 1.0178x over previous
import functools

import jax
import jax.numpy as jnp
from jax import lax
from jax.experimental import pallas as pl
from jax.experimental.pallas import tpu as pltpu
from jax.experimental.pallas import tpu_sc as plsc

B = 16384
D = 64
OUT_D = 4 * D
NUM_CORES = 2
NUM_SUBCORES = 16
NW = NUM_CORES * NUM_SUBCORES
BPW = B // NW
G = 16
NGROUP = BPW // G


@functools.partial(
    pl.kernel,
    mesh=plsc.VectorSubcoreMesh(core_axis_name="c", subcore_axis_name="s"),
    out_type=jax.ShapeDtypeStruct((B, D), jnp.float32),
    scratch_types=[
        pltpu.VMEM((BPW,), jnp.int32),
        pltpu.VMEM((2 * G, 8, D), jnp.float32),
        pltpu.VMEM((BPW, D), jnp.float32),
        pltpu.SemaphoreType.DMA,
    ],
)
def _sc_gather(idx_hbm, table3, out, idx_v, tiles_v, rows_v, sem):
    wid = lax.axis_index("s") * NUM_CORES + lax.axis_index("c")
    base = wid * BPW
    pltpu.sync_copy(idx_hbm.at[pl.ds(base, BPW)], idx_v)

    def issue_group(g, hb):
        vec = idx_v[pl.ds(g * G, G)]
        tvec = lax.shift_right_logical(vec, 3)
        for j in range(G):
            pltpu.async_copy(
                table3.at[pl.ds(tvec[j], 1)], tiles_v.at[pl.ds(hb + j, 1)],
                sem)

    issue_group(0, 0)

    def group_body(g, _):
        gbase = g * G
        hb = (g & 1) * G
        for j in range(G):
            pltpu.make_async_copy(
                table3.at[pl.ds(0, 1)], tiles_v.at[pl.ds(hb + j, 1)],
                sem).wait()

        @pl.when(g + 1 < NGROUP)
        def _():
            issue_group(g + 1, G - hb)

        vec = idx_v[pl.ds(gbase, G)]
        rvec = vec & 7
        for j in range(G):
            for k in range(D // 16):
                rows_v[gbase + j, pl.ds(k * 16, 16)] = (
                    tiles_v[hb + j, rvec[j], pl.ds(k * 16, 16)])
        return 0

    lax.fori_loop(0, NGROUP, group_body, 0)
    pltpu.sync_copy(rows_v, out.at[pl.ds(base, BPW)])


def _concat_body(z_src_ref, z_dst_ref, pos_ref, t_ref, out_ref):
    out_ref[...] = jnp.concatenate(
        [z_src_ref[...], z_dst_ref[...], pos_ref[...], t_ref[...]], axis=-1)


_R = 2048
_concat = pl.pallas_call(
    _concat_body,
    grid=(B // _R,),
    in_specs=[pl.BlockSpec((_R, D), lambda i: (i, 0))] * 4,
    out_specs=pl.BlockSpec((_R, OUT_D), lambda i: (i, 0)),
    out_shape=jax.ShapeDtypeStruct((B, OUT_D), jnp.float32),
)


def kernel(z_src, z_dst, raw_msg, t_enc, embedding_weight):
    idx = raw_msg.astype(jnp.int32)
    table3 = embedding_weight.reshape(125000, 8, D)
    pos_msg = _sc_gather(idx, table3)
    return _concat(z_src, z_dst, pos_msg, t_enc)
